# baseline (speedup 1.0000x reference)
import functools

import jax
import jax.numpy as jnp
from jax import lax
from jax.experimental import pallas as pl
from jax.experimental.pallas import tpu as pltpu
from jax.experimental.pallas import tpu_sc as plsc

N = 10000
E = 320000
D = 128

NC = 2
NS = 16
NW = NC * NS
EPT = E // NW
C = 125
NCHUNK = EPT // C
NBUF = 2
NG = NCHUNK // NBUF
NSLOT = 4
RPT = 624
TAIL_OFF = RPT * NS
TAIL = N - TAIL_OFF
DEGW = 16


def _tile_rows(s, fn):
    r0 = pl.multiple_of(s * RPT, 8)
    fn(r0, RPT)

    @pl.when(s == NS - 1)
    def _():
        fn(TAIL_OFF, TAIL)


def _agg_body(with_deg, h_hbm, idx_hbm, z_hbm, *args):
    args = list(args)
    if with_deg:
        z16_hbm, agg_out, deg_out = args[:3]
        args = args[3:]
    else:
        (agg_out,) = args[:1]
        args = args[1:]
    idx4 = args[0]
    bufs = args[1:1 + NBUF]
    rest = args[1 + NBUF:]
    if with_deg:
        ones_v = rest[0]
        gsems = rest[1:1 + NBUF]
        isems = rest[1 + NBUF:1 + NBUF + NSLOT]
        agg_sh, deg_sh = rest[1 + NBUF + NSLOT:]
    else:
        gsems = rest[:NBUF]
        isems = rest[NBUF:NBUF + NSLOT]
        (agg_sh,) = rest[NBUF + NSLOT:]

    c = lax.axis_index("c")
    s = lax.axis_index("s")
    w = c * NS + s

    def zero(r0, n):
        pltpu.sync_copy(z_hbm.at[pl.ds(r0, n)], agg_sh.at[pl.ds(r0, n)])
        if with_deg:
            pltpu.sync_copy(z16_hbm.at[pl.ds(r0, n)],
                            deg_sh.at[pl.ds(r0, n)])
    _tile_rows(s, zero)

    def stage(g, slot):
        pltpu.async_copy(idx_hbm.at[w, pl.ds(g * NBUF, NBUF)],
                         idx4.at[slot], isems[slot])

    def iwait(slot):
        pltpu.make_async_copy(idx_hbm.at[w, pl.ds(0, NBUF)],
                              idx4.at[slot], isems[slot]).wait()

    def gather(slot, b, buf, sem):
        pltpu.async_copy(h_hbm.at[idx4.at[slot, b, 0]], buf, sem,
                         priority=b % 2)

    def gwait(buf, sem):
        pltpu.make_async_copy(h_hbm.at[idx4.at[0, 0, 0]], buf, sem).wait()

    def scatter(slot, b, buf):
        pltpu.sync_copy(buf, agg_sh.at[idx4.at[slot, b, 1]], add=True)
        if with_deg:
            pltpu.sync_copy(ones_v, deg_sh.at[idx4.at[slot, b, 1]],
                            add=True)

    if with_deg:
        def fill(i, carry):
            ones_v[i, :] = jnp.ones((16,), jnp.float32)
            return carry
        lax.fori_loop(0, C, fill, 0)

    stage(0, 0)
    stage(1, 1)
    stage(2, 2)
    iwait(0)
    for b in range(NBUF):
        gather(0, b, bufs[b], gsems[b])

    plsc.subcore_barrier()

    def quad(q, carry):
        for t in range(NSLOT):
            g = q * NSLOT + t

            @pl.when(g + 1 < NG)
            def _():
                iwait((t + 1) % NSLOT)

            @pl.when(g + 3 < NG)
            def _():
                stage(g + 3, (t + 3) % NSLOT)

            for b in range(NBUF):
                gwait(bufs[b], gsems[b])
                scatter(t, b, bufs[b])

                @pl.when(g + 1 < NG)
                def _():
                    gather((t + 1) % NSLOT, b, bufs[b], gsems[b])
        return carry
    lax.fori_loop(0, NG // NSLOT, quad, 0)

    plsc.subcore_barrier()

    def out(r0, n):
        pltpu.sync_copy(agg_sh.at[pl.ds(r0, n)], agg_out.at[c, pl.ds(r0, n)])
        if with_deg:
            pltpu.sync_copy(deg_sh.at[pl.ds(r0, n)],
                            deg_out.at[c, pl.ds(r0, n)])
    _tile_rows(s, out)


_SC_MESH = plsc.VectorSubcoreMesh(core_axis_name="c", subcore_axis_name="s")
_SC_PARAMS = pltpu.CompilerParams(use_tc_tiling_on_sc=False)

_agg_with_deg = pl.kernel(
    functools.partial(_agg_body, True),
    compiler_params=_SC_PARAMS,
    out_type=(jax.ShapeDtypeStruct((NC, N, D), jnp.float32),
              jax.ShapeDtypeStruct((NC, N, DEGW), jnp.float32)),
    mesh=_SC_MESH,
    scratch_types=(
        pltpu.VMEM((NSLOT, NBUF, 2, C), jnp.int32),
        *[pltpu.VMEM((C, D), jnp.float32) for _ in range(NBUF)],
        pltpu.VMEM((C, DEGW), jnp.float32),
        *[pltpu.SemaphoreType.DMA for _ in range(NBUF)],
        *[pltpu.SemaphoreType.DMA for _ in range(NSLOT)],
        pltpu.VMEM_SHARED((N, D), jnp.float32),
        pltpu.VMEM_SHARED((N, DEGW), jnp.float32),
    ),
)

_agg_only = pl.kernel(
    functools.partial(_agg_body, False),
    compiler_params=_SC_PARAMS,
    out_type=(jax.ShapeDtypeStruct((NC, N, D), jnp.float32),),
    mesh=_SC_MESH,
    scratch_types=(
        pltpu.VMEM((NSLOT, NBUF, 2, C), jnp.int32),
        *[pltpu.VMEM((C, D), jnp.float32) for _ in range(NBUF)],
        *[pltpu.SemaphoreType.DMA for _ in range(NBUF)],
        *[pltpu.SemaphoreType.DMA for _ in range(NSLOT)],
        pltpu.VMEM_SHARED((N, D), jnp.float32),
    ),
)


def _dense_body(relu, a0, a1, d0, d1, h, A, B, cvec, out):
    deg = jnp.maximum(d0[:, 0:1] + d1[:, 0:1], 1.0)
    agg = (a0[...] + a1[...]) / deg
    y = (jnp.dot(agg, A[...], preferred_element_type=jnp.float32)
         + jnp.dot(h[...], B[...], preferred_element_type=jnp.float32)
         + cvec[...])
    if relu:
        y = jnp.maximum(y, 0.0)
    out[...] = y


_RB = 1000


def _dense_layer(a0, a1, d0, d1, h, A, B, cvec, relu):
    return pl.pallas_call(
        functools.partial(_dense_body, relu),
        grid=(N // _RB,),
        in_specs=[
            pl.BlockSpec((_RB, D), lambda i: (i, 0)),
            pl.BlockSpec((_RB, D), lambda i: (i, 0)),
            pl.BlockSpec((_RB, DEGW), lambda i: (i, 0)),
            pl.BlockSpec((_RB, DEGW), lambda i: (i, 0)),
            pl.BlockSpec((_RB, D), lambda i: (i, 0)),
            pl.BlockSpec((D, D), lambda i: (0, 0)),
            pl.BlockSpec((D, D), lambda i: (0, 0)),
            pl.BlockSpec((1, D), lambda i: (0, 0)),
        ],
        out_specs=pl.BlockSpec((_RB, D), lambda i: (i, 0)),
        out_shape=jax.ShapeDtypeStruct((N, D), jnp.float32),
    )(a0, a1, d0, d1, h, A, B, cvec)


def kernel(x, edge_index, W_l0, b_l0, W_r0, bn_g0, bn_b0, bn_rm0, bn_rv0,
           W_l1, b_l1, W_r1, bn_g1, bn_b1, bn_rm1, bn_rv1, W_l2, b_l2, W_r2):
    eidx = jnp.stack([edge_index[0].reshape(NW, NCHUNK, C),
                      edge_index[1].reshape(NW, NCHUNK, C)], axis=2)
    z = jnp.zeros((N, D), jnp.float32)
    z16 = jnp.zeros((N, DEGW), jnp.float32)

    s0 = bn_g0 / jnp.sqrt(bn_rv0 + 1e-5)
    A0 = W_l0 * s0[None, :]
    B0 = W_r0 * s0[None, :]
    c0 = ((b_l0 - bn_rm0) * s0 + bn_b0).reshape(1, D)
    s1 = bn_g1 / jnp.sqrt(bn_rv1 + 1e-5)
    A1 = W_l1 * s1[None, :]
    B1 = W_r1 * s1[None, :]
    c1 = ((b_l1 - bn_rm1) * s1 + bn_b1).reshape(1, D)
    c2 = b_l2.reshape(1, D)

    aggp, degp = _agg_with_deg(x, eidx, z, z16)
    d0, d1 = degp[0], degp[1]
    h1 = _dense_layer(aggp[0], aggp[1], d0, d1, x, A0, B0, c0, True)
    (aggp1,) = _agg_only(h1, eidx, z)
    h2 = _dense_layer(aggp1[0], aggp1[1], d0, d1, h1, A1, B1, c1, True)
    (aggp2,) = _agg_only(h2, eidx, z)
    h3 = _dense_layer(aggp2[0], aggp2[1], d0, d1, h2, W_l2, W_r2, c2, False)
    return h3


# speedup vs baseline: 1.0012x; 1.0012x over previous
import functools

import jax
import jax.numpy as jnp
from jax import lax
from jax.experimental import pallas as pl
from jax.experimental.pallas import tpu as pltpu
from jax.experimental.pallas import tpu_sc as plsc

N = 10000
E = 320000
D = 128

NC = 2
NS = 16
NW = NC * NS
EPT = E // NW
C = 125
NCHUNK = EPT // C
NBUF = 2
NG = NCHUNK // NBUF
NSLOT = 4
RPT = 624
TAIL_OFF = RPT * NS
TAIL = N - TAIL_OFF
DEGW = 16


def _tile_rows(s, fn):
    r0 = pl.multiple_of(s * RPT, 8)
    fn(r0, RPT)

    @pl.when(s == NS - 1)
    def _():
        fn(TAIL_OFF, TAIL)


def _agg_body(with_deg, h_hbm, idx_hbm, z_hbm, *args):
    args = list(args)
    if with_deg:
        z16_hbm, agg_out, deg_out = args[:3]
        args = args[3:]
    else:
        (agg_out,) = args[:1]
        args = args[1:]
    idx4 = args[0]
    bufs = args[1:1 + NBUF]
    rest = args[1 + NBUF:]
    if with_deg:
        ones_v = rest[0]
        gsems = rest[1:1 + NBUF]
        isems = rest[1 + NBUF:1 + NBUF + NSLOT]
        agg_sh, deg_sh = rest[1 + NBUF + NSLOT:]
    else:
        gsems = rest[:NBUF]
        isems = rest[NBUF:NBUF + NSLOT]
        (agg_sh,) = rest[NBUF + NSLOT:]

    c = lax.axis_index("c")
    s = lax.axis_index("s")
    w = c * NS + s

    def zero(r0, n):
        pltpu.sync_copy(z_hbm.at[pl.ds(r0, n)], agg_sh.at[pl.ds(r0, n)])
        if with_deg:
            pltpu.sync_copy(z16_hbm.at[pl.ds(r0, n)],
                            deg_sh.at[pl.ds(r0, n)])
    _tile_rows(s, zero)

    def stage(g, slot):
        pltpu.async_copy(idx_hbm.at[w, pl.ds(g * NBUF, NBUF)],
                         idx4.at[slot], isems[slot])

    def iwait(slot):
        pltpu.make_async_copy(idx_hbm.at[w, pl.ds(0, NBUF)],
                              idx4.at[slot], isems[slot]).wait()

    def gather(slot, b, buf, sem):
        pltpu.async_copy(h_hbm.at[idx4.at[slot, b, 0]], buf, sem)

    def gwait(buf, sem):
        pltpu.make_async_copy(h_hbm.at[idx4.at[0, 0, 0]], buf, sem).wait()

    def scatter(slot, b, buf):
        pltpu.sync_copy(buf, agg_sh.at[idx4.at[slot, b, 1]], add=True)
        if with_deg:
            pltpu.sync_copy(ones_v, deg_sh.at[idx4.at[slot, b, 1]],
                            add=True)

    if with_deg:
        def fill(i, carry):
            ones_v[i, :] = jnp.ones((16,), jnp.float32)
            return carry
        lax.fori_loop(0, C, fill, 0)

    stage(0, 0)
    stage(1, 1)
    stage(2, 2)
    iwait(0)
    for b in range(NBUF):
        gather(0, b, bufs[b], gsems[b])

    plsc.subcore_barrier()

    def quad(q, carry):
        for t in range(NSLOT):
            g = q * NSLOT + t

            @pl.when(g + 1 < NG)
            def _():
                iwait((t + 1) % NSLOT)

            @pl.when(g + 3 < NG)
            def _():
                stage(g + 3, (t + 3) % NSLOT)

            for b in range(NBUF):
                gwait(bufs[b], gsems[b])
                scatter(t, b, bufs[b])

                @pl.when(g + 1 < NG)
                def _():
                    gather((t + 1) % NSLOT, b, bufs[b], gsems[b])
        return carry
    lax.fori_loop(0, NG // NSLOT, quad, 0)

    plsc.subcore_barrier()

    def out(r0, n):
        pltpu.sync_copy(agg_sh.at[pl.ds(r0, n)], agg_out.at[c, pl.ds(r0, n)])
        if with_deg:
            pltpu.sync_copy(deg_sh.at[pl.ds(r0, n)],
                            deg_out.at[c, pl.ds(r0, n)])
    _tile_rows(s, out)


_SC_MESH = plsc.VectorSubcoreMesh(core_axis_name="c", subcore_axis_name="s")
_SC_PARAMS = pltpu.CompilerParams(use_tc_tiling_on_sc=False)

_agg_with_deg = pl.kernel(
    functools.partial(_agg_body, True),
    compiler_params=_SC_PARAMS,
    out_type=(jax.ShapeDtypeStruct((NC, N, D), jnp.float32),
              jax.ShapeDtypeStruct((NC, N, DEGW), jnp.float32)),
    mesh=_SC_MESH,
    scratch_types=(
        pltpu.VMEM((NSLOT, NBUF, 2, C), jnp.int32),
        *[pltpu.VMEM((C, D), jnp.float32) for _ in range(NBUF)],
        pltpu.VMEM((C, DEGW), jnp.float32),
        *[pltpu.SemaphoreType.DMA for _ in range(NBUF)],
        *[pltpu.SemaphoreType.DMA for _ in range(NSLOT)],
        pltpu.VMEM_SHARED((N, D), jnp.float32),
        pltpu.VMEM_SHARED((N, DEGW), jnp.float32),
    ),
)

_agg_only = pl.kernel(
    functools.partial(_agg_body, False),
    compiler_params=_SC_PARAMS,
    out_type=(jax.ShapeDtypeStruct((NC, N, D), jnp.float32),),
    mesh=_SC_MESH,
    scratch_types=(
        pltpu.VMEM((NSLOT, NBUF, 2, C), jnp.int32),
        *[pltpu.VMEM((C, D), jnp.float32) for _ in range(NBUF)],
        *[pltpu.SemaphoreType.DMA for _ in range(NBUF)],
        *[pltpu.SemaphoreType.DMA for _ in range(NSLOT)],
        pltpu.VMEM_SHARED((N, D), jnp.float32),
    ),
)


def _dense_body(relu, a0, a1, d0, d1, h, A, B, cvec, out):
    deg = jnp.maximum(d0[:, 0:1] + d1[:, 0:1], 1.0)
    agg = (a0[...] + a1[...]) / deg
    y = (jnp.dot(agg, A[...], preferred_element_type=jnp.float32)
         + jnp.dot(h[...], B[...], preferred_element_type=jnp.float32)
         + cvec[...])
    if relu:
        y = jnp.maximum(y, 0.0)
    out[...] = y


_RB = 1000


def _dense_layer(a0, a1, d0, d1, h, A, B, cvec, relu):
    return pl.pallas_call(
        functools.partial(_dense_body, relu),
        grid=(N // _RB,),
        in_specs=[
            pl.BlockSpec((_RB, D), lambda i: (i, 0)),
            pl.BlockSpec((_RB, D), lambda i: (i, 0)),
            pl.BlockSpec((_RB, DEGW), lambda i: (i, 0)),
            pl.BlockSpec((_RB, DEGW), lambda i: (i, 0)),
            pl.BlockSpec((_RB, D), lambda i: (i, 0)),
            pl.BlockSpec((D, D), lambda i: (0, 0)),
            pl.BlockSpec((D, D), lambda i: (0, 0)),
            pl.BlockSpec((1, D), lambda i: (0, 0)),
        ],
        out_specs=pl.BlockSpec((_RB, D), lambda i: (i, 0)),
        out_shape=jax.ShapeDtypeStruct((N, D), jnp.float32),
    )(a0, a1, d0, d1, h, A, B, cvec)


def kernel(x, edge_index, W_l0, b_l0, W_r0, bn_g0, bn_b0, bn_rm0, bn_rv0,
           W_l1, b_l1, W_r1, bn_g1, bn_b1, bn_rm1, bn_rv1, W_l2, b_l2, W_r2):
    eidx = jnp.stack([edge_index[0].reshape(NW, NCHUNK, C),
                      edge_index[1].reshape(NW, NCHUNK, C)], axis=2)
    z = jnp.zeros((N, D), jnp.float32)
    z16 = jnp.zeros((N, DEGW), jnp.float32)

    s0 = bn_g0 / jnp.sqrt(bn_rv0 + 1e-5)
    A0 = W_l0 * s0[None, :]
    B0 = W_r0 * s0[None, :]
    c0 = ((b_l0 - bn_rm0) * s0 + bn_b0).reshape(1, D)
    s1 = bn_g1 / jnp.sqrt(bn_rv1 + 1e-5)
    A1 = W_l1 * s1[None, :]
    B1 = W_r1 * s1[None, :]
    c1 = ((b_l1 - bn_rm1) * s1 + bn_b1).reshape(1, D)
    c2 = b_l2.reshape(1, D)

    aggp, degp = _agg_with_deg(x, eidx, z, z16)
    d0, d1 = degp[0], degp[1]
    h1 = _dense_layer(aggp[0], aggp[1], d0, d1, x, A0, B0, c0, True)
    (aggp1,) = _agg_only(h1, eidx, z)
    h2 = _dense_layer(aggp1[0], aggp1[1], d0, d1, h1, A1, B1, c1, True)
    (aggp2,) = _agg_only(h2, eidx, z)
    h3 = _dense_layer(aggp2[0], aggp2[1], d0, d1, h2, W_l2, W_r2, c2, False)
    return h3


# speedup vs baseline: 1.0020x; 1.0008x over previous
import functools

import jax
import jax.numpy as jnp
from jax import lax
from jax.experimental import pallas as pl
from jax.experimental.pallas import tpu as pltpu
from jax.experimental.pallas import tpu_sc as plsc

N = 10000
E = 320000
D = 128

NC = 2
NS = 16
NW = NC * NS
EPT = E // NW
C = 125
NCHUNK = EPT // C
NBUF = 2
NG = NCHUNK // NBUF
NSLOT = 4
RPT = 624
TAIL_OFF = RPT * NS
TAIL = N - TAIL_OFF
DEGW = 16


def _tile_rows(s, fn):
    r0 = pl.multiple_of(s * RPT, 8)
    fn(r0, RPT)

    @pl.when(s == NS - 1)
    def _():
        fn(TAIL_OFF, TAIL)


def _agg_body(with_deg, h_hbm, idx_hbm, z_hbm, *args):
    args = list(args)
    if with_deg:
        z16_hbm, agg_out, deg_out = args[:3]
        args = args[3:]
    else:
        (agg_out,) = args[:1]
        args = args[1:]
    idx4 = args[0]
    bufs = args[1:1 + NBUF]
    rest = args[1 + NBUF:]
    if with_deg:
        ones_v = rest[0]
        gsems = rest[1:1 + NBUF]
        isems = rest[1 + NBUF:1 + NBUF + NSLOT]
        agg_sh, deg_sh = rest[1 + NBUF + NSLOT:]
    else:
        gsems = rest[:NBUF]
        isems = rest[NBUF:NBUF + NSLOT]
        (agg_sh,) = rest[NBUF + NSLOT:]

    c = lax.axis_index("c")
    s = lax.axis_index("s")
    w = c * NS + s

    def zero(r0, n):
        pltpu.sync_copy(z_hbm.at[pl.ds(r0, n)], agg_sh.at[pl.ds(r0, n)])
        if with_deg:
            pltpu.sync_copy(z16_hbm.at[pl.ds(r0, n)],
                            deg_sh.at[pl.ds(r0, n)])
    _tile_rows(s, zero)

    def stage(g, slot):
        pltpu.async_copy(idx_hbm.at[w, pl.ds(g * NBUF, NBUF)],
                         idx4.at[slot], isems[slot])

    def iwait(slot):
        pltpu.make_async_copy(idx_hbm.at[w, pl.ds(0, NBUF)],
                              idx4.at[slot], isems[slot]).wait()

    CH = 64

    def gather(slot, b, buf, sem):
        pltpu.async_copy(h_hbm.at[idx4.at[slot, b, 0, pl.ds(0, CH)]],
                         buf.at[pl.ds(0, CH)], sem)
        pltpu.async_copy(h_hbm.at[idx4.at[slot, b, 0, pl.ds(CH, C - CH)]],
                         buf.at[pl.ds(CH, C - CH)], sem)

    def gwait(buf, sem):
        pltpu.make_async_copy(h_hbm.at[idx4.at[0, 0, 0, pl.ds(0, CH)]],
                              buf.at[pl.ds(0, CH)], sem).wait()
        pltpu.make_async_copy(h_hbm.at[idx4.at[0, 0, 0, pl.ds(CH, C - CH)]],
                              buf.at[pl.ds(CH, C - CH)], sem).wait()

    def scatter(slot, b, buf):
        pltpu.sync_copy(buf, agg_sh.at[idx4.at[slot, b, 1]], add=True)
        if with_deg:
            pltpu.sync_copy(ones_v, deg_sh.at[idx4.at[slot, b, 1]],
                            add=True)

    if with_deg:
        def fill(i, carry):
            ones_v[i, :] = jnp.ones((16,), jnp.float32)
            return carry
        lax.fori_loop(0, C, fill, 0)

    stage(0, 0)
    stage(1, 1)
    stage(2, 2)
    iwait(0)
    for b in range(NBUF):
        gather(0, b, bufs[b], gsems[b])

    plsc.subcore_barrier()

    def quad(q, carry):
        for t in range(NSLOT):
            g = q * NSLOT + t

            @pl.when(g + 1 < NG)
            def _():
                iwait((t + 1) % NSLOT)

            @pl.when(g + 3 < NG)
            def _():
                stage(g + 3, (t + 3) % NSLOT)

            for b in range(NBUF):
                gwait(bufs[b], gsems[b])
                scatter(t, b, bufs[b])

                @pl.when(g + 1 < NG)
                def _():
                    gather((t + 1) % NSLOT, b, bufs[b], gsems[b])
        return carry
    lax.fori_loop(0, NG // NSLOT, quad, 0)

    plsc.subcore_barrier()

    def out(r0, n):
        pltpu.sync_copy(agg_sh.at[pl.ds(r0, n)], agg_out.at[c, pl.ds(r0, n)])
        if with_deg:
            pltpu.sync_copy(deg_sh.at[pl.ds(r0, n)],
                            deg_out.at[c, pl.ds(r0, n)])
    _tile_rows(s, out)


_SC_MESH = plsc.VectorSubcoreMesh(core_axis_name="c", subcore_axis_name="s")
_SC_PARAMS = pltpu.CompilerParams(use_tc_tiling_on_sc=False)

_agg_with_deg = pl.kernel(
    functools.partial(_agg_body, True),
    compiler_params=_SC_PARAMS,
    out_type=(jax.ShapeDtypeStruct((NC, N, D), jnp.float32),
              jax.ShapeDtypeStruct((NC, N, DEGW), jnp.float32)),
    mesh=_SC_MESH,
    scratch_types=(
        pltpu.VMEM((NSLOT, NBUF, 2, C), jnp.int32),
        *[pltpu.VMEM((C, D), jnp.float32) for _ in range(NBUF)],
        pltpu.VMEM((C, DEGW), jnp.float32),
        *[pltpu.SemaphoreType.DMA for _ in range(NBUF)],
        *[pltpu.SemaphoreType.DMA for _ in range(NSLOT)],
        pltpu.VMEM_SHARED((N, D), jnp.float32),
        pltpu.VMEM_SHARED((N, DEGW), jnp.float32),
    ),
)

_agg_only = pl.kernel(
    functools.partial(_agg_body, False),
    compiler_params=_SC_PARAMS,
    out_type=(jax.ShapeDtypeStruct((NC, N, D), jnp.float32),),
    mesh=_SC_MESH,
    scratch_types=(
        pltpu.VMEM((NSLOT, NBUF, 2, C), jnp.int32),
        *[pltpu.VMEM((C, D), jnp.float32) for _ in range(NBUF)],
        *[pltpu.SemaphoreType.DMA for _ in range(NBUF)],
        *[pltpu.SemaphoreType.DMA for _ in range(NSLOT)],
        pltpu.VMEM_SHARED((N, D), jnp.float32),
    ),
)


def _dense_body(relu, a0, a1, d0, d1, h, A, B, cvec, out):
    deg = jnp.maximum(d0[:, 0:1] + d1[:, 0:1], 1.0)
    agg = (a0[...] + a1[...]) / deg
    y = (jnp.dot(agg, A[...], preferred_element_type=jnp.float32)
         + jnp.dot(h[...], B[...], preferred_element_type=jnp.float32)
         + cvec[...])
    if relu:
        y = jnp.maximum(y, 0.0)
    out[...] = y


_RB = 1000


def _dense_layer(a0, a1, d0, d1, h, A, B, cvec, relu):
    return pl.pallas_call(
        functools.partial(_dense_body, relu),
        grid=(N // _RB,),
        in_specs=[
            pl.BlockSpec((_RB, D), lambda i: (i, 0)),
            pl.BlockSpec((_RB, D), lambda i: (i, 0)),
            pl.BlockSpec((_RB, DEGW), lambda i: (i, 0)),
            pl.BlockSpec((_RB, DEGW), lambda i: (i, 0)),
            pl.BlockSpec((_RB, D), lambda i: (i, 0)),
            pl.BlockSpec((D, D), lambda i: (0, 0)),
            pl.BlockSpec((D, D), lambda i: (0, 0)),
            pl.BlockSpec((1, D), lambda i: (0, 0)),
        ],
        out_specs=pl.BlockSpec((_RB, D), lambda i: (i, 0)),
        out_shape=jax.ShapeDtypeStruct((N, D), jnp.float32),
    )(a0, a1, d0, d1, h, A, B, cvec)


def kernel(x, edge_index, W_l0, b_l0, W_r0, bn_g0, bn_b0, bn_rm0, bn_rv0,
           W_l1, b_l1, W_r1, bn_g1, bn_b1, bn_rm1, bn_rv1, W_l2, b_l2, W_r2):
    eidx = jnp.stack([edge_index[0].reshape(NW, NCHUNK, C),
                      edge_index[1].reshape(NW, NCHUNK, C)], axis=2)
    z = jnp.zeros((N, D), jnp.float32)
    z16 = jnp.zeros((N, DEGW), jnp.float32)

    s0 = bn_g0 / jnp.sqrt(bn_rv0 + 1e-5)
    A0 = W_l0 * s0[None, :]
    B0 = W_r0 * s0[None, :]
    c0 = ((b_l0 - bn_rm0) * s0 + bn_b0).reshape(1, D)
    s1 = bn_g1 / jnp.sqrt(bn_rv1 + 1e-5)
    A1 = W_l1 * s1[None, :]
    B1 = W_r1 * s1[None, :]
    c1 = ((b_l1 - bn_rm1) * s1 + bn_b1).reshape(1, D)
    c2 = b_l2.reshape(1, D)

    aggp, degp = _agg_with_deg(x, eidx, z, z16)
    d0, d1 = degp[0], degp[1]
    h1 = _dense_layer(aggp[0], aggp[1], d0, d1, x, A0, B0, c0, True)
    (aggp1,) = _agg_only(h1, eidx, z)
    h2 = _dense_layer(aggp1[0], aggp1[1], d0, d1, h1, A1, B1, c1, True)
    (aggp2,) = _agg_only(h2, eidx, z)
    h3 = _dense_layer(aggp2[0], aggp2[1], d0, d1, h2, W_l2, W_r2, c2, False)
    return h3
